# baseline (device time: 13123 ns/iter reference)
import jax
import jax.numpy as jnp
from jax import lax
from jax.experimental import pallas as pl
from jax.experimental.pallas import tpu as pltpu

N_DEV = 8
N_PEERS = N_DEV - 1
BLOCK_ROWS = 256


def kernel(x, dy, gamma):
    del gamma
    m_per, d = x.shape
    n_blocks = m_per // BLOCK_ROWS
    assert m_per % BLOCK_ROWS == 0

    def body(x_hbm, dy_hbm, out_ref, xb, dyb, send_ref, recv_ref,
             copy_sems, send_sems, recv_sems):
        my_pos = lax.axis_index("i")
        peers = [(my_pos + k) % N_DEV for k in range(1, N_DEV)]

        barrier_sem = pltpu.get_barrier_semaphore()
        for p in peers:
            pl.semaphore_signal(
                barrier_sem, inc=1,
                device_id=(p,), device_id_type=pl.DeviceIdType.MESH,
            )

        def start_block_copy(b, slot):
            cx = pltpu.make_async_copy(
                x_hbm.at[pl.ds(b * BLOCK_ROWS, BLOCK_ROWS), :],
                xb.at[slot], copy_sems.at[slot, 0])
            cy = pltpu.make_async_copy(
                dy_hbm.at[pl.ds(b * BLOCK_ROWS, BLOCK_ROWS), :],
                dyb.at[slot], copy_sems.at[slot, 1])
            cx.start()
            cy.start()
            return cx, cy

        inflight = start_block_copy(0, 0)
        dgamma = jnp.zeros((d,), jnp.float32)
        dbeta = jnp.zeros((d,), jnp.float32)
        for b in range(n_blocks):
            slot = b % 2
            nxt = None
            if b + 1 < n_blocks:
                nxt = start_block_copy(b + 1, (b + 1) % 2)
            inflight[0].wait()
            inflight[1].wait()
            inflight = nxt

            xv = xb[slot, :, :]
            dyv = dyb[slot, :, :]
            s1 = jnp.sum(xv, axis=1, keepdims=True)
            s2 = jnp.sum(xv * xv, axis=1, keepdims=True)
            mu = s1 * (1.0 / d)
            var = s2 * (1.0 / d) - mu * mu
            rstd = lax.rsqrt(var + 1e-5)
            dgamma = dgamma + jnp.sum(dyv * ((xv - mu) * rstd), axis=0)
            dbeta = dbeta + jnp.sum(dyv, axis=0)

        acc = jnp.stack([dgamma, dbeta])
        send_ref[:, :] = acc

        pl.semaphore_wait(barrier_sem, N_PEERS)

        rdmas = []
        for k in range(1, N_DEV):
            rdma = pltpu.make_async_remote_copy(
                src_ref=send_ref,
                dst_ref=recv_ref.at[k - 1],
                send_sem=send_sems.at[k - 1],
                recv_sem=recv_sems.at[k - 1],
                device_id=(peers[k - 1],),
                device_id_type=pl.DeviceIdType.MESH,
            )
            rdma.start()
            rdmas.append(rdma)

        for k, rdma in enumerate(rdmas):
            rdma.wait()
            acc = acc + recv_ref[k, :, :]

        out_ref[:, :] = acc

    return pl.pallas_call(
        body,
        out_shape=jax.ShapeDtypeStruct((2, d), jnp.float32),
        in_specs=[
            pl.BlockSpec(memory_space=pl.ANY),
            pl.BlockSpec(memory_space=pl.ANY),
        ],
        out_specs=pl.BlockSpec(memory_space=pltpu.VMEM),
        scratch_shapes=[
            pltpu.VMEM((2, BLOCK_ROWS, d), jnp.float32),
            pltpu.VMEM((2, BLOCK_ROWS, d), jnp.float32),
            pltpu.VMEM((2, d), jnp.float32),
            pltpu.VMEM((N_PEERS, 2, d), jnp.float32),
            pltpu.SemaphoreType.DMA((2, 2)),
            pltpu.SemaphoreType.DMA((N_PEERS,)),
            pltpu.SemaphoreType.DMA((N_PEERS,)),
        ],
        compiler_params=pltpu.CompilerParams(collective_id=0),
    )(x, dy)


# device time: 12659 ns/iter; 1.0367x vs baseline; 1.0367x over previous
import jax
import jax.numpy as jnp
from jax import lax
from jax.experimental import pallas as pl
from jax.experimental.pallas import tpu as pltpu

N_DEV = 8
N_PEERS = N_DEV - 1
N_HALVES = 2


def kernel(x, dy, gamma):
    del gamma
    m_per, d = x.shape
    rows_half = m_per // N_HALVES

    def half_partial(x_ref, dy_ref, h):
        xv = x_ref[pl.ds(h * rows_half, rows_half), :]
        dyv = dy_ref[pl.ds(h * rows_half, rows_half), :]
        s1 = jnp.sum(xv, axis=1, keepdims=True)
        s2 = jnp.sum(xv * xv, axis=1, keepdims=True)
        mu = s1 * (1.0 / d)
        var = s2 * (1.0 / d) - mu * mu
        rstd = lax.rsqrt(var + 1e-5)
        dgamma = jnp.sum(dyv * ((xv - mu) * rstd), axis=0)
        dbeta = jnp.sum(dyv, axis=0)
        return jnp.stack([dgamma, dbeta])

    def body(x_ref, dy_ref, out_ref, send_ref, recv_ref, send_sems, recv_sems):
        my_pos = lax.axis_index("i")
        peers = [(my_pos + k) % N_DEV for k in range(1, N_DEV)]

        barrier_sem = pltpu.get_barrier_semaphore()
        for p in peers:
            pl.semaphore_signal(
                barrier_sem, inc=1,
                device_id=(p,), device_id_type=pl.DeviceIdType.MESH,
            )

        def broadcast(h):
            rdmas = []
            for k in range(1, N_DEV):
                rdma = pltpu.make_async_remote_copy(
                    src_ref=send_ref.at[h],
                    dst_ref=recv_ref.at[h, k - 1],
                    send_sem=send_sems.at[h, k - 1],
                    recv_sem=recv_sems.at[h, k - 1],
                    device_id=(peers[k - 1],),
                    device_id_type=pl.DeviceIdType.MESH,
                )
                rdma.start()
                rdmas.append(rdma)
            return rdmas

        acc = half_partial(x_ref, dy_ref, 0)
        send_ref[0, :, :] = acc
        pl.semaphore_wait(barrier_sem, N_PEERS)
        rdmas0 = broadcast(0)

        acc = acc + half_partial(x_ref, dy_ref, 1)
        send_ref[1, :, :] = acc - send_ref[0, :, :]
        rdmas1 = broadcast(1)

        for h, rdmas in enumerate((rdmas0, rdmas1)):
            for k, rdma in enumerate(rdmas):
                rdma.wait()
                acc = acc + recv_ref[h, k, :, :]

        out_ref[:, :] = acc

    return pl.pallas_call(
        body,
        out_shape=jax.ShapeDtypeStruct((2, d), jnp.float32),
        in_specs=[
            pl.BlockSpec(memory_space=pltpu.VMEM),
            pl.BlockSpec(memory_space=pltpu.VMEM),
        ],
        out_specs=pl.BlockSpec(memory_space=pltpu.VMEM),
        scratch_shapes=[
            pltpu.VMEM((N_HALVES, 2, d), jnp.float32),
            pltpu.VMEM((N_HALVES, N_PEERS, 2, d), jnp.float32),
            pltpu.SemaphoreType.DMA((N_HALVES, N_PEERS)),
            pltpu.SemaphoreType.DMA((N_HALVES, N_PEERS)),
        ],
        compiler_params=pltpu.CompilerParams(collective_id=0),
    )(x, dy)
